# baseline (speedup 1.0000x reference)
import functools

import jax
import jax.numpy as jnp
from jax import lax
from jax.experimental import pallas as pl
from jax.experimental.pallas import tpu as pltpu
from jax.experimental.pallas import tpu_sc as plsc

NC = 2
NS = 16
G = 128
NB = 2
CH = 8
FRAC0 = 1.0
NW = NC * NS


def _copy_rows(copy_one, sid):
  def run(n):
    rpt = (n // NS) & ~7
    tail = n - NS * rpt
    copy_one(sid * rpt, rpt)
    if tail:
      @pl.when(sid == NS - 1)
      def _():
        copy_one(NS * rpt, tail)
  return run


def _make_deg_kernel(n, groups):
  gpt = groups // NW
  mesh = plsc.VectorSubcoreMesh(
      core_axis_name="c", subcore_axis_name="s", num_cores=NC, num_subcores=NS)

  @functools.partial(
      pl.kernel,
      out_type=jax.ShapeDtypeStruct((NC * n, 128), jnp.float32),
      mesh=mesh,
      scratch_types=[
          pltpu.VMEM((gpt, 2, G), jnp.int32),
          pltpu.VMEM((G, 128), jnp.float32),
          pltpu.VMEM_SHARED((n + 8, 128), jnp.float32),
          pltpu.SemaphoreType.DMA,
      ],
  )
  def deg_kernel(edges_hbm, zeros_hbm, ones_hbm, out_hbm, edgev, onesv, acc,
                 sem):
    cid = lax.axis_index("c")
    sid = lax.axis_index("s")
    wid = cid * NS + sid

    def init(off, cnt):
      pltpu.sync_copy(zeros_hbm.at[pl.ds(off, cnt)], acc.at[pl.ds(off, cnt)])

    _copy_rows(init, sid)(n)
    pltpu.sync_copy(ones_hbm, onesv)
    pltpu.sync_copy(edges_hbm.at[pl.ds(wid * gpt, gpt)], edgev)
    plsc.subcore_barrier()

    FK = 8

    def body(i, carry):
      for b in range(FK):
        pltpu.async_copy(onesv, acc.at[edgev.at[i * FK + b, 1]], sem,
                         add=True)
      for b in range(FK):
        pltpu.make_async_copy(onesv, acc.at[edgev.at[i * FK + b, 1]],
                              sem).wait()
      return carry

    lax.fori_loop(0, gpt // FK, body, 0)
    plsc.subcore_barrier()

    def readback(off, cnt):
      pltpu.sync_copy(acc.at[pl.ds(off, cnt)],
                      out_hbm.at[pl.ds(cid * n + off, cnt)])

    _copy_rows(readback, sid)(n)

  return deg_kernel


def _make_agg_kernel(n, d, groups):
  gps = groups // NS
  gpt0 = int(round(gps * FRAC0 / (2 * CH))) * 2 * CH
  gpt1 = gps - gpt0
  assert gpt0 % (2 * CH) == 0 and gpt1 % (2 * CH) == 0
  gpc0, gpc1 = gpt0 // CH, gpt1 // CH
  mesh = plsc.VectorSubcoreMesh(
      core_axis_name="c", subcore_axis_name="s", num_cores=NC, num_subcores=NS)

  @functools.partial(
      pl.kernel,
      out_type=jax.ShapeDtypeStruct((NC * n, d), jnp.float32),
      mesh=mesh,
      scratch_types=[
          pltpu.VMEM((2, CH, 2, G), jnp.int32),
          pltpu.VMEM((NB, G, d), jnp.float32),
          pltpu.VMEM_SHARED((n + 8, d), jnp.float32),
          pltpu.SemaphoreType.DMA((2,)),
          pltpu.SemaphoreType.DMA((NB,)),
          pltpu.SemaphoreType.DMA((NB,)),
      ],
  )
  def agg_kernel(y_hbm, zeros_hbm, edges_hbm, out_hbm,
                 idxv, rowsv, acc, isem, gsem, ssem):
    cid = lax.axis_index("c")
    sid = lax.axis_index("s")
    base = jnp.where(cid == 0, sid * gpt0, NS * gpt0 + sid * gpt1)
    trips = jnp.where(cid == 0, gpc0 // 2, gpc1 // 2)
    gpc = 2 * trips

    def init(off, cnt):
      @pl.when(cid == 0)
      def _():
        pltpu.sync_copy(y_hbm.at[pl.ds(off, cnt)], acc.at[pl.ds(off, cnt)])

      @pl.when(cid == 1)
      def _():
        pltpu.sync_copy(zeros_hbm.at[pl.ds(off, cnt)],
                        acc.at[pl.ds(off, cnt)])

    _copy_rows(init, sid)(n)

    @pl.when(trips > 0)
    def _():
      pltpu.async_copy(edges_hbm.at[pl.ds(base, CH)], idxv.at[0],
                       isem.at[0])

    plsc.subcore_barrier()

    def chunk(c, cb):
      pltpu.make_async_copy(
          edges_hbm.at[pl.ds(base, CH)], idxv.at[cb], isem.at[cb]).wait()

      @pl.when(c + 1 < gpc)
      def _():
        pltpu.async_copy(edges_hbm.at[pl.ds(base + (c + 1) * CH, CH)],
                         idxv.at[1 - cb], isem.at[1 - cb])

      for b in range(min(NB, CH)):
        pltpu.async_copy(y_hbm.at[idxv.at[cb, b, 0]], rowsv.at[b],
                         gsem.at[b])
      for j in range(CH):
        b = j % NB
        pltpu.make_async_copy(y_hbm.at[idxv.at[cb, j, 0]], rowsv.at[b],
                              gsem.at[b]).wait()
        pltpu.async_copy(rowsv.at[b], acc.at[idxv.at[cb, j, 1]],
                         ssem.at[b], add=True)
        pltpu.make_async_copy(rowsv.at[b], acc.at[idxv.at[cb, j, 1]],
                              ssem.at[b]).wait()
        if j + NB < CH:
          pltpu.async_copy(y_hbm.at[idxv.at[cb, j + NB, 0]], rowsv.at[b],
                           gsem.at[b])

    def body(i, carry):
      chunk(2 * i, 0)
      chunk(2 * i + 1, 1)
      return carry

    lax.fori_loop(0, trips, body, 0)
    plsc.subcore_barrier()

    def readback(off, cnt):
      pltpu.sync_copy(acc.at[pl.ds(off, cnt)],
                      out_hbm.at[pl.ds(cid * n + off, cnt)])

    _copy_rows(readback, sid)(n)

  return agg_kernel


def _tc_first(degp, x, w):
  n, d_in = x.shape
  d_out = w.shape[1]

  def body(degp_ref, x_ref, w_ref, dinv_ref, y_ref):
    deg = degp_ref[0:n, 0:1] + degp_ref[n:, 0:1] + 1.0
    dinv = lax.rsqrt(deg)
    dinv_ref[...] = dinv
    xw = jnp.dot(x_ref[...], w_ref[...], preferred_element_type=jnp.float32)
    y_ref[...] = dinv * xw

  return pl.pallas_call(
      body,
      out_shape=(
          jax.ShapeDtypeStruct((n, 1), jnp.float32),
          jax.ShapeDtypeStruct((n, d_out), jnp.float32),
      ),
  )(degp, x, w)


def _tc_mid(p, dinv, b, w):
  n = dinv.shape[0]
  d_out = w.shape[1]

  def body(p_ref, dinv_ref, b_ref, w_ref, y_ref):
    dinv = dinv_ref[...]
    h = jnp.maximum(dinv * (p_ref[0:n, :] + p_ref[n:, :]) + b_ref[...], 0.0)
    y_ref[...] = dinv * jnp.dot(h, w_ref[...],
                                preferred_element_type=jnp.float32)

  return pl.pallas_call(
      body,
      out_shape=jax.ShapeDtypeStruct((n, d_out), jnp.float32),
  )(p, dinv, b.reshape(1, -1), w)


def _tc_last(q, dinv, b):
  n = dinv.shape[0]
  d = b.shape[0]

  def body(q_ref, dinv_ref, b_ref, out_ref):
    out_ref[...] = (dinv_ref[...] * (q_ref[0:n, 0:d] + q_ref[n:, 0:d])
                    + b_ref[...])

  return pl.pallas_call(
      body,
      out_shape=jax.ShapeDtypeStruct((n, d), jnp.float32),
  )(q, dinv, b.reshape(1, -1))


def kernel(features, edge_index, W1, b1, W2, b2, W3, b3):
  n, _ = features.shape
  e = edge_index.shape[1]
  assert n % NS == 0

  groups = -(-e // (G * NW * 8)) * (NW * 8)
  pad = groups * G - e
  src_p = jnp.concatenate(
      [edge_index[0], jnp.zeros((pad,), jnp.int32)]).reshape(groups, 1, G)
  dst_p = jnp.concatenate(
      [edge_index[1], jnp.full((pad,), n, jnp.int32)]).reshape(groups, 1, G)
  edges_p = jnp.concatenate([src_p, dst_p], axis=1)

  ones = jnp.ones((G, 128), jnp.float32)
  d_hid = W1.shape[1]
  zeros_hid = jnp.zeros((n, d_hid), jnp.float32)
  W3p = jnp.pad(W3, ((0, 0), (0, d_hid - W3.shape[1])))

  deg_k = _make_deg_kernel(n, groups)
  agg_hid = _make_agg_kernel(n, d_hid, groups)

  degp = deg_k(edges_p, zeros_hid, ones)
  dinv, y1 = _tc_first(degp, features, W1)
  p1 = agg_hid(y1, zeros_hid, edges_p)
  y2 = _tc_mid(p1, dinv, b1, W2)
  p2 = agg_hid(y2, zeros_hid, edges_p)
  y3 = _tc_mid(p2, dinv, b2, W3p)
  q = agg_hid(y3, zeros_hid, edges_p)
  return _tc_last(q, dinv, b3)


# speedup vs baseline: 1.3047x; 1.3047x over previous
import functools

import jax
import jax.numpy as jnp
from jax import lax
from jax.experimental import pallas as pl
from jax.experimental.pallas import tpu as pltpu
from jax.experimental.pallas import tpu_sc as plsc

NC = 2
NS = 16
G = 128
NB = 2
CH = 8
FRAC0 = 0.9
NW = NC * NS


def _copy_rows(copy_one, sid):
  def run(n):
    rpt = (n // NS) & ~7
    tail = n - NS * rpt
    copy_one(sid * rpt, rpt)
    if tail:
      @pl.when(sid == NS - 1)
      def _():
        copy_one(NS * rpt, tail)
  return run


def _make_deg_kernel(n, groups):
  gpt = groups // NW
  mesh = plsc.VectorSubcoreMesh(
      core_axis_name="c", subcore_axis_name="s", num_cores=NC, num_subcores=NS)

  @functools.partial(
      pl.kernel,
      out_type=jax.ShapeDtypeStruct((NC * n, 128), jnp.float32),
      mesh=mesh,
      scratch_types=[
          pltpu.VMEM((gpt, 2, G), jnp.int32),
          pltpu.VMEM((G, 128), jnp.float32),
          pltpu.VMEM_SHARED((n + 8, 128), jnp.float32),
          pltpu.SemaphoreType.DMA,
      ],
  )
  def deg_kernel(edges_hbm, zeros_hbm, ones_hbm, out_hbm, edgev, onesv, acc,
                 sem):
    cid = lax.axis_index("c")
    sid = lax.axis_index("s")
    wid = cid * NS + sid

    def init(off, cnt):
      pltpu.sync_copy(zeros_hbm.at[pl.ds(off, cnt)], acc.at[pl.ds(off, cnt)])

    _copy_rows(init, sid)(n)
    pltpu.sync_copy(ones_hbm, onesv)
    pltpu.sync_copy(edges_hbm.at[pl.ds(wid * gpt, gpt)], edgev)
    plsc.subcore_barrier()

    FK = 8

    def body(i, carry):
      for b in range(FK):
        pltpu.async_copy(onesv, acc.at[edgev.at[i * FK + b, 1]], sem,
                         add=True)
      for b in range(FK):
        pltpu.make_async_copy(onesv, acc.at[edgev.at[i * FK + b, 1]],
                              sem).wait()
      return carry

    lax.fori_loop(0, gpt // FK, body, 0)
    plsc.subcore_barrier()

    def readback(off, cnt):
      pltpu.sync_copy(acc.at[pl.ds(off, cnt)],
                      out_hbm.at[pl.ds(cid * n + off, cnt)])

    _copy_rows(readback, sid)(n)

  return deg_kernel


def _make_agg_kernel(n, d, groups):
  gps = groups // NS
  gpt0 = int(round(gps * FRAC0 / (2 * CH))) * 2 * CH
  gpt1 = gps - gpt0
  assert gpt0 % (2 * CH) == 0 and gpt1 % (2 * CH) == 0
  gpc0, gpc1 = gpt0 // CH, gpt1 // CH
  mesh = plsc.VectorSubcoreMesh(
      core_axis_name="c", subcore_axis_name="s", num_cores=NC, num_subcores=NS)

  @functools.partial(
      pl.kernel,
      out_type=jax.ShapeDtypeStruct((NC * n, d), jnp.float32),
      mesh=mesh,
      scratch_types=[
          pltpu.VMEM((2, CH, 2, G), jnp.int32),
          pltpu.VMEM((NB, G, d), jnp.float32),
          pltpu.VMEM_SHARED((n + 8, d), jnp.float32),
          pltpu.SemaphoreType.DMA((2,)),
          pltpu.SemaphoreType.DMA((NB,)),
          pltpu.SemaphoreType.DMA((NB,)),
      ],
  )
  def agg_kernel(y_hbm, zeros_hbm, edges_hbm, out_hbm,
                 idxv, rowsv, acc, isem, gsem, ssem):
    cid = lax.axis_index("c")
    sid = lax.axis_index("s")
    base = jnp.where(cid == 0, sid * gpt0, NS * gpt0 + sid * gpt1)
    trips = jnp.where(cid == 0, gpc0 // 2, gpc1 // 2)
    gpc = 2 * trips

    def init(off, cnt):
      @pl.when(cid == 0)
      def _():
        pltpu.sync_copy(y_hbm.at[pl.ds(off, cnt)], acc.at[pl.ds(off, cnt)])

      @pl.when(cid == 1)
      def _():
        pltpu.sync_copy(zeros_hbm.at[pl.ds(off, cnt)],
                        acc.at[pl.ds(off, cnt)])

    _copy_rows(init, sid)(n)

    @pl.when(trips > 0)
    def _():
      pltpu.async_copy(edges_hbm.at[pl.ds(base, CH)], idxv.at[0],
                       isem.at[0])

    plsc.subcore_barrier()

    @pl.when(trips > 0)
    def _():
      pltpu.make_async_copy(edges_hbm.at[pl.ds(base, CH)], idxv.at[0],
                            isem.at[0]).wait()

      @pl.when(1 < gpc)
      def _():
        pltpu.async_copy(edges_hbm.at[pl.ds(base + CH, CH)], idxv.at[1],
                         isem.at[1])

      for b in range(min(NB, CH)):
        pltpu.async_copy(y_hbm.at[idxv.at[0, b, 0]], rowsv.at[b],
                         gsem.at[b])

    def chunk(c, cb):
      for j in range(CH):
        b = j % NB
        pltpu.make_async_copy(y_hbm.at[idxv.at[cb, j, 0]], rowsv.at[b],
                              gsem.at[b]).wait()
        pltpu.async_copy(rowsv.at[b], acc.at[idxv.at[cb, j, 1]],
                         ssem.at[b], add=True)
        pltpu.make_async_copy(rowsv.at[b], acc.at[idxv.at[cb, j, 1]],
                              ssem.at[b]).wait()
        jn = j + NB
        if jn < CH:
          pltpu.async_copy(y_hbm.at[idxv.at[cb, jn, 0]], rowsv.at[b],
                           gsem.at[b])
        else:
          if jn == CH:
            @pl.when(c + 1 < gpc)
            def _():
              pltpu.make_async_copy(
                  edges_hbm.at[pl.ds(base, CH)], idxv.at[1 - cb],
                  isem.at[1 - cb]).wait()

          @pl.when(c + 1 < gpc)
          def _():
            pltpu.async_copy(y_hbm.at[idxv.at[1 - cb, jn - CH, 0]],
                             rowsv.at[b], gsem.at[b])

      @pl.when(c + 2 < gpc)
      def _():
        pltpu.async_copy(edges_hbm.at[pl.ds(base + (c + 2) * CH, CH)],
                         idxv.at[cb], isem.at[cb])

    def body(i, carry):
      chunk(2 * i, 0)
      chunk(2 * i + 1, 1)
      return carry

    lax.fori_loop(0, trips, body, 0)
    plsc.subcore_barrier()

    def readback(off, cnt):
      pltpu.sync_copy(acc.at[pl.ds(off, cnt)],
                      out_hbm.at[pl.ds(cid * n + off, cnt)])

    _copy_rows(readback, sid)(n)

  return agg_kernel


def _tc_first(degp, x, w):
  n, d_in = x.shape
  d_out = w.shape[1]

  def body(degp_ref, x_ref, w_ref, dinv_ref, y_ref):
    deg = degp_ref[0:n, 0:1] + degp_ref[n:, 0:1] + 1.0
    dinv = lax.rsqrt(deg)
    dinv_ref[...] = dinv
    xw = jnp.dot(x_ref[...], w_ref[...], preferred_element_type=jnp.float32)
    y_ref[...] = dinv * xw

  return pl.pallas_call(
      body,
      out_shape=(
          jax.ShapeDtypeStruct((n, 1), jnp.float32),
          jax.ShapeDtypeStruct((n, d_out), jnp.float32),
      ),
  )(degp, x, w)


def _tc_mid(p, dinv, b, w):
  n = dinv.shape[0]
  d_out = w.shape[1]

  def body(p_ref, dinv_ref, b_ref, w_ref, y_ref):
    dinv = dinv_ref[...]
    h = jnp.maximum(dinv * (p_ref[0:n, :] + p_ref[n:, :]) + b_ref[...], 0.0)
    y_ref[...] = dinv * jnp.dot(h, w_ref[...],
                                preferred_element_type=jnp.float32)

  return pl.pallas_call(
      body,
      out_shape=jax.ShapeDtypeStruct((n, d_out), jnp.float32),
  )(p, dinv, b.reshape(1, -1), w)


def _tc_last(q, dinv, b):
  n = dinv.shape[0]
  d = b.shape[0]

  def body(q_ref, dinv_ref, b_ref, out_ref):
    out_ref[...] = (dinv_ref[...] * (q_ref[0:n, 0:d] + q_ref[n:, 0:d])
                    + b_ref[...])

  return pl.pallas_call(
      body,
      out_shape=jax.ShapeDtypeStruct((n, d), jnp.float32),
  )(q, dinv, b.reshape(1, -1))


def kernel(features, edge_index, W1, b1, W2, b2, W3, b3):
  n, _ = features.shape
  e = edge_index.shape[1]
  assert n % NS == 0

  groups = -(-e // (G * NW * 8)) * (NW * 8)
  pad = groups * G - e
  src_p = jnp.concatenate(
      [edge_index[0], jnp.zeros((pad,), jnp.int32)]).reshape(groups, 1, G)
  dst_p = jnp.concatenate(
      [edge_index[1], jnp.full((pad,), n, jnp.int32)]).reshape(groups, 1, G)
  edges_p = jnp.concatenate([src_p, dst_p], axis=1)

  ones = jnp.ones((G, 128), jnp.float32)
  d_hid = W1.shape[1]
  zeros_hid = jnp.zeros((n, d_hid), jnp.float32)
  W3p = jnp.pad(W3, ((0, 0), (0, d_hid - W3.shape[1])))

  deg_k = _make_deg_kernel(n, groups)
  agg_hid = _make_agg_kernel(n, d_hid, groups)

  degp = deg_k(edges_p, zeros_hid, ones)
  dinv, y1 = _tc_first(degp, features, W1)
  p1 = agg_hid(y1, zeros_hid, edges_p)
  y2 = _tc_mid(p1, dinv, b1, W2)
  p2 = agg_hid(y2, zeros_hid, edges_p)
  y3 = _tc_mid(p2, dinv, b2, W3p)
  q = agg_hid(y3, zeros_hid, edges_p)
  return _tc_last(q, dinv, b3)


# speedup vs baseline: 1.3132x; 1.0065x over previous
import functools

import jax
import jax.numpy as jnp
from jax import lax
from jax.experimental import pallas as pl
from jax.experimental.pallas import tpu as pltpu
from jax.experimental.pallas import tpu_sc as plsc

NC = 2
NS = 16
G = 128
NB = 2
CH = 4
FRAC0 = 0.95
NW = NC * NS


def _copy_rows(copy_one, sid):
  def run(n):
    rpt = (n // NS) & ~7
    tail = n - NS * rpt
    copy_one(sid * rpt, rpt)
    if tail:
      @pl.when(sid == NS - 1)
      def _():
        copy_one(NS * rpt, tail)
  return run


def _make_deg_kernel(n, groups):
  gpt = groups // NW
  mesh = plsc.VectorSubcoreMesh(
      core_axis_name="c", subcore_axis_name="s", num_cores=NC, num_subcores=NS)

  @functools.partial(
      pl.kernel,
      out_type=jax.ShapeDtypeStruct((NC * n, 128), jnp.float32),
      mesh=mesh,
      scratch_types=[
          pltpu.VMEM((gpt, 2, G), jnp.int32),
          pltpu.VMEM((G, 128), jnp.float32),
          pltpu.VMEM_SHARED((n + 8, 128), jnp.float32),
          pltpu.SemaphoreType.DMA,
      ],
  )
  def deg_kernel(edges_hbm, zeros_hbm, ones_hbm, out_hbm, edgev, onesv, acc,
                 sem):
    cid = lax.axis_index("c")
    sid = lax.axis_index("s")
    wid = cid * NS + sid

    def init(off, cnt):
      pltpu.sync_copy(zeros_hbm.at[pl.ds(off, cnt)], acc.at[pl.ds(off, cnt)])

    _copy_rows(init, sid)(n)
    pltpu.sync_copy(ones_hbm, onesv)
    pltpu.sync_copy(edges_hbm.at[pl.ds(wid * gpt, gpt)], edgev)
    plsc.subcore_barrier()

    FK = 8

    def body(i, carry):
      for b in range(FK):
        pltpu.async_copy(onesv, acc.at[edgev.at[i * FK + b, 1]], sem,
                         add=True)
      for b in range(FK):
        pltpu.make_async_copy(onesv, acc.at[edgev.at[i * FK + b, 1]],
                              sem).wait()
      return carry

    lax.fori_loop(0, gpt // FK, body, 0)
    plsc.subcore_barrier()

    def readback(off, cnt):
      pltpu.sync_copy(acc.at[pl.ds(off, cnt)],
                      out_hbm.at[pl.ds(cid * n + off, cnt)])

    _copy_rows(readback, sid)(n)

  return deg_kernel


def _make_agg_kernel(n, d, groups):
  gps = groups // NS
  gpt0 = int(round(gps * FRAC0 / (2 * CH))) * 2 * CH
  gpt1 = gps - gpt0
  assert gpt0 % (2 * CH) == 0 and gpt1 % (2 * CH) == 0
  gpc0, gpc1 = gpt0 // CH, gpt1 // CH
  mesh = plsc.VectorSubcoreMesh(
      core_axis_name="c", subcore_axis_name="s", num_cores=NC, num_subcores=NS)

  @functools.partial(
      pl.kernel,
      out_type=jax.ShapeDtypeStruct((NC * n, d), jnp.float32),
      mesh=mesh,
      scratch_types=[
          pltpu.VMEM((2, CH, 2, G), jnp.int32),
          pltpu.VMEM((NB, G, d), jnp.float32),
          pltpu.VMEM_SHARED((n + 8, d), jnp.float32),
          pltpu.SemaphoreType.DMA((2,)),
          pltpu.SemaphoreType.DMA((NB,)),
          pltpu.SemaphoreType.DMA((NB,)),
      ],
  )
  def agg_kernel(y_hbm, zeros_hbm, edges_hbm, out_hbm,
                 idxv, rowsv, acc, isem, gsem, ssem):
    cid = lax.axis_index("c")
    sid = lax.axis_index("s")
    base = jnp.where(cid == 0, sid * gpt0, NS * gpt0 + sid * gpt1)
    trips = jnp.where(cid == 0, gpc0 // 2, gpc1 // 2)
    gpc = 2 * trips

    def init(off, cnt):
      @pl.when(cid == 0)
      def _():
        pltpu.sync_copy(y_hbm.at[pl.ds(off, cnt)], acc.at[pl.ds(off, cnt)])

      @pl.when(cid == 1)
      def _():
        pltpu.sync_copy(zeros_hbm.at[pl.ds(off, cnt)],
                        acc.at[pl.ds(off, cnt)])

    _copy_rows(init, sid)(n)

    @pl.when(trips > 0)
    def _():
      pltpu.async_copy(edges_hbm.at[pl.ds(base, CH)], idxv.at[0],
                       isem.at[0])

    plsc.subcore_barrier()

    @pl.when(trips > 0)
    def _():
      pltpu.make_async_copy(edges_hbm.at[pl.ds(base, CH)], idxv.at[0],
                            isem.at[0]).wait()

      @pl.when(1 < gpc)
      def _():
        pltpu.async_copy(edges_hbm.at[pl.ds(base + CH, CH)], idxv.at[1],
                         isem.at[1])

      for b in range(min(NB, CH)):
        pltpu.async_copy(y_hbm.at[idxv.at[0, b, 0]], rowsv.at[b],
                         gsem.at[b])

    def chunk(c, cb):
      for j in range(CH):
        b = j % NB
        pltpu.make_async_copy(y_hbm.at[idxv.at[cb, j, 0]], rowsv.at[b],
                              gsem.at[b]).wait()
        pltpu.async_copy(rowsv.at[b], acc.at[idxv.at[cb, j, 1]],
                         ssem.at[b], add=True)
        pltpu.make_async_copy(rowsv.at[b], acc.at[idxv.at[cb, j, 1]],
                              ssem.at[b]).wait()
        jn = j + NB
        if jn < CH:
          pltpu.async_copy(y_hbm.at[idxv.at[cb, jn, 0]], rowsv.at[b],
                           gsem.at[b])
        else:
          if jn == CH:
            @pl.when(c + 1 < gpc)
            def _():
              pltpu.make_async_copy(
                  edges_hbm.at[pl.ds(base, CH)], idxv.at[1 - cb],
                  isem.at[1 - cb]).wait()

          @pl.when(c + 1 < gpc)
          def _():
            pltpu.async_copy(y_hbm.at[idxv.at[1 - cb, jn - CH, 0]],
                             rowsv.at[b], gsem.at[b])

      @pl.when(c + 2 < gpc)
      def _():
        pltpu.async_copy(edges_hbm.at[pl.ds(base + (c + 2) * CH, CH)],
                         idxv.at[cb], isem.at[cb])

    def body(i, carry):
      chunk(2 * i, 0)
      chunk(2 * i + 1, 1)
      return carry

    lax.fori_loop(0, trips, body, 0)
    plsc.subcore_barrier()

    def readback(off, cnt):
      pltpu.sync_copy(acc.at[pl.ds(off, cnt)],
                      out_hbm.at[pl.ds(cid * n + off, cnt)])

    _copy_rows(readback, sid)(n)

  return agg_kernel


def _tc_first(degp, x, w):
  n, d_in = x.shape
  d_out = w.shape[1]

  def body(degp_ref, x_ref, w_ref, dinv_ref, y_ref):
    deg = degp_ref[0:n, 0:1] + degp_ref[n:, 0:1] + 1.0
    dinv = lax.rsqrt(deg)
    dinv_ref[...] = dinv
    xw = jnp.dot(x_ref[...], w_ref[...], preferred_element_type=jnp.float32)
    y_ref[...] = dinv * xw

  return pl.pallas_call(
      body,
      out_shape=(
          jax.ShapeDtypeStruct((n, 1), jnp.float32),
          jax.ShapeDtypeStruct((n, d_out), jnp.float32),
      ),
  )(degp, x, w)


def _tc_mid(p, dinv, b, w):
  n = dinv.shape[0]
  d_out = w.shape[1]

  def body(p_ref, dinv_ref, b_ref, w_ref, y_ref):
    dinv = dinv_ref[...]
    h = jnp.maximum(dinv * (p_ref[0:n, :] + p_ref[n:, :]) + b_ref[...], 0.0)
    y_ref[...] = dinv * jnp.dot(h, w_ref[...],
                                preferred_element_type=jnp.float32)

  return pl.pallas_call(
      body,
      out_shape=jax.ShapeDtypeStruct((n, d_out), jnp.float32),
  )(p, dinv, b.reshape(1, -1), w)


def _tc_last(q, dinv, b):
  n = dinv.shape[0]
  d = b.shape[0]

  def body(q_ref, dinv_ref, b_ref, out_ref):
    out_ref[...] = (dinv_ref[...] * (q_ref[0:n, 0:d] + q_ref[n:, 0:d])
                    + b_ref[...])

  return pl.pallas_call(
      body,
      out_shape=jax.ShapeDtypeStruct((n, d), jnp.float32),
  )(q, dinv, b.reshape(1, -1))


def kernel(features, edge_index, W1, b1, W2, b2, W3, b3):
  n, _ = features.shape
  e = edge_index.shape[1]
  assert n % NS == 0

  groups = -(-e // (G * NW * 8)) * (NW * 8)
  pad = groups * G - e
  src_p = jnp.concatenate(
      [edge_index[0], jnp.zeros((pad,), jnp.int32)]).reshape(groups, 1, G)
  dst_p = jnp.concatenate(
      [edge_index[1], jnp.full((pad,), n, jnp.int32)]).reshape(groups, 1, G)
  edges_p = jnp.concatenate([src_p, dst_p], axis=1)

  ones = jnp.ones((G, 128), jnp.float32)
  d_hid = W1.shape[1]
  zeros_hid = jnp.zeros((n, d_hid), jnp.float32)
  W3p = jnp.pad(W3, ((0, 0), (0, d_hid - W3.shape[1])))

  deg_k = _make_deg_kernel(n, groups)
  agg_hid = _make_agg_kernel(n, d_hid, groups)

  degp = deg_k(edges_p, zeros_hid, ones)
  dinv, y1 = _tc_first(degp, features, W1)
  p1 = agg_hid(y1, zeros_hid, edges_p)
  y2 = _tc_mid(p1, dinv, b1, W2)
  p2 = agg_hid(y2, zeros_hid, edges_p)
  y3 = _tc_mid(p2, dinv, b2, W3p)
  q = agg_hid(y3, zeros_hid, edges_p)
  return _tc_last(q, dinv, b3)
